# baseline (speedup 1.0000x reference)
import functools

import jax
import jax.numpy as jnp
from jax import lax
from jax.experimental import pallas as pl
from jax.experimental.pallas import tpu as pltpu
from jax.experimental.pallas import tpu_sc as plsc

NC = 2
NS = 16
NW = NC * NS
L = 16

NM, NU = 20000, 5000
NM_PAD, NU_PAD = 20480, 5120
CM, CU = NM_PAD // NW, NU_PAD // NW
VM, VU = CM // L, CU // L
GRP = 2 * CM + 2 * CU
GRP_PAD = 1664
RUN = 2 * GRP_PAD

_F = jnp.float32
_I = jnp.int32


def _build():
  mesh = plsc.VectorSubcoreMesh(
      core_axis_name="c", subcore_axis_name="s",
      num_cores=NC, num_subcores=NS)

  @functools.partial(
      pl.kernel,
      out_type=jax.ShapeDtypeStruct((NW, L), _F),
      mesh=mesh,
      scratch_types=[pltpu.VMEM((RUN,), _I),
                     pltpu.VMEM((GRP_PAD,), _F), pltpu.VMEM((GRP_PAD,), _F),
                     pltpu.VMEM((L,), _F), pltpu.SemaphoreType.DMA],
  )
  def run(pred_hbm, tgt_hbm, civ_hbm, out_hbm, civ, vp, vt, acc_v, sem):
    wid = lax.axis_index("s") * NC + lax.axis_index("c")
    lanes = lax.iota(_I, L)

    pltpu.async_copy(civ_hbm.at[pl.ds(wid * RUN, RUN)], civ, sem).wait()
    gp = pltpu.async_copy(pred_hbm.at[civ.at[pl.ds(0, GRP_PAD)]], vp, sem)
    gt = pltpu.async_copy(tgt_hbm.at[civ.at[pl.ds(GRP_PAD, GRP_PAD)]], vt, sem)
    gp.wait()
    gt.wait()

    def term(va, oa, vb, ob, nvec, ch, n_real):
      base = wid * ch
      def body(j, acc):
        o = j * L
        d = va[pl.ds(oa + o, L)] - vb[pl.ds(ob + o, L)]
        pos = base + o + lanes
        return acc + jnp.where(pos < n_real, d * d, jnp.zeros_like(d))
      return lax.fori_loop(0, nvec, body, jnp.zeros((L,), _F), unroll=4)

    t_b = term(vp, 0, vt, 0, VM, CM, NM)
    t_d = term(vp, CM, vt, CM, VM, CM, NM)
    t_up = term(vp, 2 * CM, vp, 2 * CM + CU, VU, CU, NU)
    t_ut = term(vt, 2 * CM, vt, 2 * CM + CU, VU, CU, NU)
    acc_v[...] = 2.0 * (t_b + t_d) + (t_up + t_ut)
    pltpu.sync_copy(acc_v, out_hbm.at[wid])

  return run


_run = _build()


def _lin(c, npad):
  i = c[:, 0] * 16384 + c[:, 1] * 128 + c[:, 2]
  return jnp.pad(i, (0, npad - i.shape[0])).reshape(NW, npad // NW)


def kernel(pred_field, tgt_field,
           matched_pred_birth, matched_pred_death,
           matched_tgt_birth, matched_tgt_death,
           unmatched_pred_birth, unmatched_pred_death,
           unmatched_tgt_birth, unmatched_tgt_death):
  z = jnp.zeros((NW, GRP_PAD - GRP), _I)
  civ = jnp.concatenate([
      _lin(matched_pred_birth, NM_PAD), _lin(matched_pred_death, NM_PAD),
      _lin(unmatched_pred_birth, NU_PAD), _lin(unmatched_pred_death, NU_PAD), z,
      _lin(matched_tgt_birth, NM_PAD), _lin(matched_tgt_death, NM_PAD),
      _lin(unmatched_tgt_birth, NU_PAD), _lin(unmatched_tgt_death, NU_PAD), z,
  ], axis=1).reshape(-1)
  out = _run(pred_field.reshape(-1), tgt_field.reshape(-1), civ)
  return jnp.sum(out).reshape(1)


# speedup vs baseline: 1.2830x; 1.2830x over previous
import functools

import jax
import jax.numpy as jnp
from jax import lax
from jax.experimental import pallas as pl
from jax.experimental.pallas import tpu as pltpu
from jax.experimental.pallas import tpu_sc as plsc

NC = 2
NS = 16
NW = NC * NS
L = 16

NM, NU = 20000, 5000
NM_PAD, NU_PAD = 20480, 5120
CM, CU = NM_PAD // NW, NU_PAD // NW
VM, VU = CM // L, CU // L
GRP = 2 * CM + 2 * CU
GRP_PAD = 1664
RUN = 2 * GRP_PAD

_F = jnp.float32
_I = jnp.int32


def _build():
  mesh = plsc.VectorSubcoreMesh(
      core_axis_name="c", subcore_axis_name="s",
      num_cores=NC, num_subcores=NS)

  @functools.partial(
      pl.kernel,
      out_type=jax.ShapeDtypeStruct((NW, L), _F),
      mesh=mesh,
      scratch_types=[pltpu.VMEM((RUN,), _I),
                     pltpu.VMEM((GRP_PAD,), _F), pltpu.VMEM((GRP_PAD,), _F),
                     pltpu.VMEM((L,), _F), pltpu.SemaphoreType.DMA],
  )
  def run(pred_hbm, tgt_hbm, civ_hbm, out_hbm, civ, vp, vt, acc_v, sem):
    wid = lax.axis_index("s") * NC + lax.axis_index("c")
    lanes = lax.iota(_I, L)

    pltpu.async_copy(civ_hbm.at[pl.ds(wid * RUN, RUN)], civ, sem).wait()
    gps = []
    for tab, vv, goff in ((pred_hbm, vp, 0), (tgt_hbm, vt, GRP_PAD)):
      for off, sz in ((0, CM), (CM, CM), (2 * CM, 2 * CU)):
        gps.append(pltpu.async_copy(
            tab.at[civ.at[pl.ds(goff + off, sz)]], vv.at[pl.ds(off, sz)], sem))
    for g in gps:
      g.wait()

    def term(va, oa, vb, ob, nvec, ch, n_real):
      base = wid * ch
      def body(j, acc):
        o = j * L
        d = va[pl.ds(oa + o, L)] - vb[pl.ds(ob + o, L)]
        pos = base + o + lanes
        return acc + jnp.where(pos < n_real, d * d, jnp.zeros_like(d))
      return lax.fori_loop(0, nvec, body, jnp.zeros((L,), _F), unroll=4)

    t_b = term(vp, 0, vt, 0, VM, CM, NM)
    t_d = term(vp, CM, vt, CM, VM, CM, NM)
    t_up = term(vp, 2 * CM, vp, 2 * CM + CU, VU, CU, NU)
    t_ut = term(vt, 2 * CM, vt, 2 * CM + CU, VU, CU, NU)
    acc_v[...] = 2.0 * (t_b + t_d) + (t_up + t_ut)
    pltpu.sync_copy(acc_v, out_hbm.at[wid])

  return run


_run = _build()


def _lin(c, npad):
  i = c[:, 0] * 16384 + c[:, 1] * 128 + c[:, 2]
  return jnp.pad(i, (0, npad - i.shape[0])).reshape(NW, npad // NW)


def kernel(pred_field, tgt_field,
           matched_pred_birth, matched_pred_death,
           matched_tgt_birth, matched_tgt_death,
           unmatched_pred_birth, unmatched_pred_death,
           unmatched_tgt_birth, unmatched_tgt_death):
  z = jnp.zeros((NW, GRP_PAD - GRP), _I)
  civ = jnp.concatenate([
      _lin(matched_pred_birth, NM_PAD), _lin(matched_pred_death, NM_PAD),
      _lin(unmatched_pred_birth, NU_PAD), _lin(unmatched_pred_death, NU_PAD), z,
      _lin(matched_tgt_birth, NM_PAD), _lin(matched_tgt_death, NM_PAD),
      _lin(unmatched_tgt_birth, NU_PAD), _lin(unmatched_tgt_death, NU_PAD), z,
  ], axis=1).reshape(-1)
  out = _run(pred_field.reshape(-1), tgt_field.reshape(-1), civ)
  return jnp.sum(out).reshape(1)


# speedup vs baseline: 1.3285x; 1.0355x over previous
import functools

import jax
import jax.numpy as jnp
from jax import lax
from jax.experimental import pallas as pl
from jax.experimental.pallas import tpu as pltpu
from jax.experimental.pallas import tpu_sc as plsc

NC = 2
NS = 16
NW = NC * NS
L = 16

NM, NU = 20000, 5000
NM_PAD, NU_PAD = 20480, 5120
CM, CU = NM_PAD // NW, NU_PAD // NW
VM, VU = CM // L, CU // L
GRP = 2 * CM + 2 * CU
GRP_PAD = 1664
RUN = 2 * GRP_PAD

_F = jnp.float32
_I = jnp.int32


def _build():
  mesh = plsc.VectorSubcoreMesh(
      core_axis_name="c", subcore_axis_name="s",
      num_cores=NC, num_subcores=NS)

  @functools.partial(
      pl.kernel,
      out_type=jax.ShapeDtypeStruct((NW, L), _F),
      mesh=mesh,
      scratch_types=[pltpu.VMEM((RUN,), _I),
                     pltpu.VMEM((GRP_PAD,), _F), pltpu.VMEM((GRP_PAD,), _F),
                     pltpu.VMEM((L,), _F), pltpu.SemaphoreType.DMA],
  )
  def run(pred_hbm, tgt_hbm, civ_hbm, out_hbm, civ, vp, vt, acc_v, sem):
    wid = lax.axis_index("s") * NC + lax.axis_index("c")
    lanes = lax.iota(_I, L)

    pltpu.async_copy(civ_hbm.at[pl.ds(wid * RUN, RUN)], civ, sem).wait()
    gps = []
    for tab, vv, goff in ((pred_hbm, vp, 0), (tgt_hbm, vt, GRP_PAD)):
      for off, sz in ((0, CM), (CM, CM), (2 * CM, 2 * CU)):
        gps.append(pltpu.async_copy(
            tab.at[civ.at[pl.ds(goff + off, sz)]], vv.at[pl.ds(off, sz)], sem))
    for g in gps:
      g.wait()

    def term(va, oa, vb, ob, nvec, ch, n_real):
      base = wid * ch
      def body(j, acc):
        o = j * L
        d = va[pl.ds(oa + o, L)] - vb[pl.ds(ob + o, L)]
        pos = base + o + lanes
        return acc + jnp.where(pos < n_real, d * d, jnp.zeros_like(d))
      return lax.fori_loop(0, nvec, body, jnp.zeros((L,), _F), unroll=4)

    t_b = term(vp, 0, vt, 0, VM, CM, NM)
    t_d = term(vp, CM, vt, CM, VM, CM, NM)
    t_up = term(vp, 2 * CM, vp, 2 * CM + CU, VU, CU, NU)
    t_ut = term(vt, 2 * CM, vt, 2 * CM + CU, VU, CU, NU)
    acc_v[...] = 2.0 * (t_b + t_d) + (t_up + t_ut)
    pltpu.sync_copy(acc_v, out_hbm.at[wid])

  return run


_run = _build()


_LIN_W = jnp.array([[16384.0], [128.0], [1.0]], jnp.float32)


def _lin(c, npad):
  i = (c.astype(jnp.float32) @ _LIN_W).astype(jnp.int32)[:, 0]
  return jnp.pad(i, (0, npad - i.shape[0])).reshape(NW, npad // NW)


def kernel(pred_field, tgt_field,
           matched_pred_birth, matched_pred_death,
           matched_tgt_birth, matched_tgt_death,
           unmatched_pred_birth, unmatched_pred_death,
           unmatched_tgt_birth, unmatched_tgt_death):
  z = jnp.zeros((NW, GRP_PAD - GRP), _I)
  civ = jnp.concatenate([
      _lin(matched_pred_birth, NM_PAD), _lin(matched_pred_death, NM_PAD),
      _lin(unmatched_pred_birth, NU_PAD), _lin(unmatched_pred_death, NU_PAD), z,
      _lin(matched_tgt_birth, NM_PAD), _lin(matched_tgt_death, NM_PAD),
      _lin(unmatched_tgt_birth, NU_PAD), _lin(unmatched_tgt_death, NU_PAD), z,
  ], axis=1).reshape(-1)
  out = _run(pred_field.reshape(-1), tgt_field.reshape(-1), civ)
  return jnp.sum(out).reshape(1)
